# baseline (speedup 1.0000x reference)
import jax
import jax.numpy as jnp
from jax import lax
from jax.experimental import pallas as pl
from jax.experimental.pallas import tpu as pltpu
from jax.experimental.pallas import tpu_sc as plsc

_EMBED_DIM = 32
_BATCH = 16384
_ROWS = 1000000

_info = plsc.get_sparse_core_info()
_NC, _NS, _L = _info.num_cores, _info.num_subcores, _info.num_lanes
_NW = _NC * _NS
_CH = 512
_CH_SHIFT = 9
_NBUF = 4
_NFULL = 1952
_TAIL0 = _NFULL * _CH
_TAIL1 = _TAIL0 + 512
_TAIL1_LEN = _ROWS - _TAIL1
_IDX_VREGS = _BATCH // _L


def _body(idx_hbm, tt_hbm, tail_hbm, out_hbm, idx_v, rlist, blist, buf,
          rowstage, sem_s, sem_o):
    w = lax.axis_index("s") * _NC + lax.axis_index("c")
    pltpu.sync_copy(idx_hbm, idx_v)

    lane_iota = lax.iota(jnp.int32, _L)
    sentinel = jnp.full((_L,), jnp.int32(2**30), jnp.int32)

    plsc.store_scatter(rlist, [lane_iota], sentinel)

    def filt(i, base):
        v = idx_v[pl.ds(i * _L, _L)]
        mask = jnp.bitwise_and(
            lax.shift_right_logical(v, _CH_SHIFT), jnp.int32(_NW - 1)
        ) == jnp.full((_L,), jnp.int32(0), jnp.int32) + w
        mi = jnp.where(mask, jnp.int32(1), jnp.int32(0))
        rank = plsc.cumsum(mi) - 1
        pos = base + rank
        plsc.store_scatter(rlist, [pos], v, mask=mask)
        plsc.store_scatter(blist, [pos], lane_iota + i * _L, mask=mask)
        cnt = plsc.all_reduce_population_count(mask)
        new_base = base + cnt
        plsc.store_scatter(rlist, [new_base + lane_iota], sentinel)
        return new_base

    base = lax.fori_loop(
        0, _IDX_VREGS, filt, jnp.zeros((_L,), jnp.int32)
    )
    count = jnp.max(base)
    nv = (count + jnp.int32(_L - 1)) // jnp.int32(_L)

    def process_chunk(lb, clen, cbuf):
        def scan_list(q, _):
            r16 = rlist[pl.ds(q * _L, _L)]
            b16 = blist[pl.ds(q * _L, _L)]
            inm = jnp.logical_and(r16 >= lb, r16 < lb + clen)

            @pl.when(jnp.any(inm))
            def _():
                local = jnp.where(inm, r16 - lb, jnp.int32(0))
                for s in range(_EMBED_DIM):
                    svec = jnp.full((_L,), jnp.int32(s), jnp.int32)
                    vals = plsc.load_gather(cbuf, [svec, local])
                    plsc.store_scatter(rowstage, [lane_iota, svec], vals)
                bm = jnp.where(inm, b16, jnp.int32(-1))
                pltpu.async_copy(
                    rowstage,
                    out_hbm.at[plsc.Indices(bm, ignored_value=-1)],
                    sem_o,
                ).wait()

            return 0

        lax.fori_loop(0, nv, scan_list, 0)

    n_rounds = _NFULL // _NW

    def lbof(t):
        return pl.multiple_of((w + t * _NW) * _CH, _CH)

    def fire(t, p):
        pltpu.async_copy(tt_hbm.at[:, pl.ds(lbof(t), _CH)], buf.at[p], sem_s)

    def step(t, p):
        @pl.when(t <= n_rounds - 1)
        def _():
            pltpu.make_async_copy(
                tt_hbm.at[:, pl.ds(0, _CH)], buf.at[p], sem_s
            ).wait()
            process_chunk(lbof(t), jnp.int32(_CH), buf.at[p])

            @pl.when(t + _NBUF <= n_rounds - 1)
            def _():
                fire(t + _NBUF, p)

    for p in range(_NBUF):
        fire(p, p)

    def ring(q, _):
        for k in range(_NBUF):
            step(q * _NBUF + k, k)
        return 0

    lax.fori_loop(0, (n_rounds + _NBUF - 1) // _NBUF, ring, 0)

    @pl.when(w == 0)
    def _():
        pltpu.sync_copy(tt_hbm.at[:, pl.ds(_TAIL0, _CH)], buf.at[0])
        process_chunk(jnp.int32(_TAIL0), jnp.int32(_CH), buf.at[0])

    @pl.when(w == 1)
    def _():
        pltpu.sync_copy(tail_hbm, buf.at[0].at[:, pl.ds(0, 128)])
        process_chunk(jnp.int32(_TAIL1), jnp.int32(_TAIL1_LEN), buf.at[0])


def kernel(num_group, table):
    idx = num_group.astype(jnp.int32)
    tt = table.T
    tail = jnp.pad(table[_TAIL1:], ((0, 128 - _TAIL1_LEN), (0, 0))).T
    k = pl.kernel(
        _body,
        out_type=jax.ShapeDtypeStruct((_BATCH, 128), jnp.float32),
        mesh=plsc.VectorSubcoreMesh(core_axis_name="c", subcore_axis_name="s"),
        scratch_types=[
            pltpu.VMEM((_BATCH,), jnp.int32),
            pltpu.VMEM((_BATCH + _L,), jnp.int32),
            pltpu.VMEM((_BATCH,), jnp.int32),
            pltpu.VMEM((_NBUF, _EMBED_DIM, _CH), jnp.float32),
            pltpu.VMEM((_L, 128), jnp.float32),
            pltpu.SemaphoreType.DMA,
            pltpu.SemaphoreType.DMA,
        ],
        compiler_params=pltpu.CompilerParams(
            use_tc_tiling_on_sc=True, needs_layout_passes=False
        ),
    )
    return k(idx, tt, tail)[:, :_EMBED_DIM]


# speedup vs baseline: 1.1648x; 1.1648x over previous
import jax
import jax.numpy as jnp
from jax import lax
from jax.experimental import pallas as pl
from jax.experimental.pallas import tpu as pltpu
from jax.experimental.pallas import tpu_sc as plsc

_EMBED_DIM = 32
_BATCH = 16384
_ROWS = 1000000

_info = plsc.get_sparse_core_info()
_NC, _NS, _L = _info.num_cores, _info.num_subcores, _info.num_lanes
_NW = _NC * _NS
_CH = 1024
_CH_SHIFT = 10
_NBUF = 2
_NFULL = 976
_TAIL0 = _NFULL * _CH
_TAIL1 = _TAIL0 + 512
_TAIL1_LEN = _ROWS - _TAIL1
_IDX_VREGS = _BATCH // _L


def _body(idx_hbm, tt_hbm, tail_hbm, out_hbm, idx_v, rlist, blist, buf,
          rowstage, sem_s, sem_o):
    w = lax.axis_index("s") * _NC + lax.axis_index("c")
    pltpu.sync_copy(idx_hbm, idx_v)

    lane_iota = lax.iota(jnp.int32, _L)
    sentinel = jnp.full((_L,), jnp.int32(2**30), jnp.int32)

    plsc.store_scatter(rlist, [lane_iota], sentinel)

    def filt(i, base):
        v = idx_v[pl.ds(i * _L, _L)]
        mask = jnp.bitwise_and(
            lax.shift_right_logical(v, _CH_SHIFT), jnp.int32(_NW - 1)
        ) == jnp.full((_L,), jnp.int32(0), jnp.int32) + w
        mi = jnp.where(mask, jnp.int32(1), jnp.int32(0))
        rank = plsc.cumsum(mi) - 1
        pos = base + rank
        plsc.store_scatter(rlist, [pos], v, mask=mask)
        plsc.store_scatter(blist, [pos], lane_iota + i * _L, mask=mask)
        cnt = plsc.all_reduce_population_count(mask)
        new_base = base + cnt
        plsc.store_scatter(rlist, [new_base + lane_iota], sentinel)
        return new_base

    base = lax.fori_loop(
        0, _IDX_VREGS, filt, jnp.zeros((_L,), jnp.int32)
    )
    count = jnp.max(base)
    nv = (count + jnp.int32(_L - 1)) // jnp.int32(_L)

    def process_chunk(lb, clen, cbuf):
        def scan_list(q, _):
            r16 = rlist[pl.ds(q * _L, _L)]
            b16 = blist[pl.ds(q * _L, _L)]
            inm = jnp.logical_and(r16 >= lb, r16 < lb + clen)

            @pl.when(jnp.any(inm))
            def _():
                local = jnp.where(inm, r16 - lb, jnp.int32(0))
                for s in range(_EMBED_DIM):
                    svec = jnp.full((_L,), jnp.int32(s), jnp.int32)
                    vals = plsc.load_gather(cbuf, [svec, local])
                    plsc.store_scatter(rowstage, [lane_iota, svec], vals)
                bm = jnp.where(inm, b16, jnp.int32(-1))
                pltpu.async_copy(
                    rowstage,
                    out_hbm.at[plsc.Indices(bm, ignored_value=-1)],
                    sem_o,
                ).wait()

            return 0

        lax.fori_loop(0, nv, scan_list, 0)

    def valid(t):
        return (w + t * _NW) <= (_NFULL - 1)

    def lbof(t):
        return pl.multiple_of((w + t * _NW) * _CH, _CH)

    def fire(t, p):
        for g in range(4):
            pltpu.async_copy(
                tt_hbm.at[pl.ds(8 * g, 8), pl.ds(lbof(t), _CH)],
                buf.at[p].at[pl.ds(8 * g, 8)],
                sem_s,
            )

    def drain(p):
        for g in range(4):
            pltpu.make_async_copy(
                tt_hbm.at[pl.ds(0, 8), pl.ds(0, _CH)],
                buf.at[p].at[pl.ds(8 * g, 8)],
                sem_s,
            ).wait()

    def step(t, p):
        @pl.when(valid(t))
        def _():
            drain(p)
            process_chunk(lbof(t), jnp.int32(_CH), buf.at[p])

            @pl.when(valid(t + _NBUF))
            def _():
                fire(t + _NBUF, p)

    for p in range(_NBUF):
        fire(p, p)

    n_rounds_max = (_NFULL + _NW - 1) // _NW

    def ring(q, _):
        for k in range(_NBUF):
            step(q * _NBUF + k, k)
        return 0

    lax.fori_loop(0, (n_rounds_max + _NBUF - 1) // _NBUF, ring, 0)

    @pl.when(w == 16)
    def _():
        pltpu.sync_copy(tt_hbm.at[:, pl.ds(_TAIL0, 512)],
                        buf.at[0].at[:, pl.ds(0, 512)])
        process_chunk(jnp.int32(_TAIL0), jnp.int32(512), buf.at[0])
        pltpu.sync_copy(tail_hbm, buf.at[0].at[:, pl.ds(0, 128)])
        process_chunk(jnp.int32(_TAIL1), jnp.int32(_TAIL1_LEN), buf.at[0])


def kernel(num_group, table):
    idx = num_group.astype(jnp.int32)
    tt = table.T
    tail = jnp.pad(table[_TAIL1:], ((0, 128 - _TAIL1_LEN), (0, 0))).T
    k = pl.kernel(
        _body,
        out_type=jax.ShapeDtypeStruct((_BATCH, 128), jnp.float32),
        mesh=plsc.VectorSubcoreMesh(core_axis_name="c", subcore_axis_name="s"),
        scratch_types=[
            pltpu.VMEM((_BATCH,), jnp.int32),
            pltpu.VMEM((_BATCH + _L,), jnp.int32),
            pltpu.VMEM((_BATCH,), jnp.int32),
            pltpu.VMEM((_NBUF, _EMBED_DIM, _CH), jnp.float32),
            pltpu.VMEM((_L, 128), jnp.float32),
            pltpu.SemaphoreType.DMA,
            pltpu.SemaphoreType.DMA,
        ],
        compiler_params=pltpu.CompilerParams(
            use_tc_tiling_on_sc=True, needs_layout_passes=False
        ),
    )
    return k(idx, tt, tail)[:, :_EMBED_DIM]
